# Initial kernel scaffold; baseline (speedup 1.0000x reference)
import functools

import jax
import jax.numpy as jnp
from jax import lax
from jax.experimental import pallas as pl
from jax.experimental.pallas import tpu as pltpu
from jax.experimental.pallas import tpu_sc as plsc

NCAM = 8
NID = 1000
NIDP = 1024
D = 512
N = 4096
MARGIN = 0.3
WARM_UP_EPOCHS = 5

FBLK = 512
SBLK = 128
T2_ROWS = N + 512
T2_BLK = T2_ROWS // NCAM


def _norm_tables_kernel(a_ref, f_ref, t1_ref, yy_ref, fn_ref, t2_ref):
    a = a_ref[0]
    ss = jnp.sum(a * a, axis=1, keepdims=True)
    an = a / (jnp.sqrt(ss) + 1e-12)
    t1_ref[0, :NID, :D] = an
    t1_ref[0, :NID, D:] = 0.5 * an
    t1_ref[0, NID:, :] = jnp.zeros((NIDP - NID, 2 * D), jnp.float32)
    anp = t1_ref[0, :, :D]
    ones = jnp.ones((1, D), jnp.float32)
    yy = lax.dot_general(ones, anp * anp, (((1,), (1,)), ((), ())),
                         preferred_element_type=jnp.float32)
    colid = lax.broadcasted_iota(jnp.int32, (1, NIDP), 1)
    yy_ref[0] = jnp.where(colid >= NID, jnp.float32(1e9), yy)
    f = f_ref[...]
    fss = jnp.sum(f * f, axis=1, keepdims=True)
    fn = f / (jnp.sqrt(fss) + 1e-12)
    fn_ref[...] = fn
    t2_ref[:FBLK, :] = 0.5 * fn
    t2_ref[FBLK:, :] = jnp.zeros((T2_BLK - FBLK, D), jnp.float32)


def _winner_kernel(k_ref, w_ref):
    b = pl.program_id(0)
    kv = (b * 128 + lax.broadcasted_iota(jnp.int32, (1, 128), 1)
          ).astype(jnp.float32)
    keys = k_ref[...]
    eq = keys == kv
    nidx = lax.broadcasted_iota(jnp.float32, (N, 128), 0)
    cand = jnp.where(eq, nidx, jnp.float32(-1.0))
    w_ref[0] = jnp.max(cand, axis=0, keepdims=True)


def _loss_kernel(t1_ref, yy_ref, f_ref, lab_ref, cam_ref, loss_ref):
    i = pl.program_id(0)
    j = pl.program_id(1)

    @pl.when(jnp.logical_and(i == 0, j == 0))
    def _():
        loss_ref[0, 0] = jnp.float32(0.0)

    f = f_ref[...]
    a = t1_ref[0, :, :]
    s = lax.dot_general(f, a, (((1,), (1,)), ((), ())),
                        preferred_element_type=jnp.float32)
    xx = jnp.sum(f * f, axis=1, keepdims=True)
    d = xx + yy_ref[0] - 2.0 * s
    d = jnp.maximum(d, jnp.float32(1e-12))
    lab = lab_ref[...]
    col = lax.broadcasted_iota(jnp.float32, (SBLK, NIDP), 1)
    oneh = col == lab
    dn = jnp.min(jnp.where(oneh, jnp.float32(1e30), d), axis=1,
                 keepdims=True)
    dp = jnp.sum(jnp.where(oneh, d, jnp.float32(0.0)), axis=1,
                 keepdims=True)
    cam = cam_ref[...]
    hinge = jnp.maximum(dp - dn + jnp.float32(MARGIN), jnp.float32(0.0))
    contrib = jnp.where(cam == i.astype(jnp.float32), hinge,
                        jnp.float32(0.0))
    loss_ref[0, 0] += jnp.sum(contrib)


def _update_kernel(win_hbm, t1_hbm, t2_hbm, out_hbm,
                   win_v, idx1_v, idx2_v, buf, sem):
    c = lax.axis_index("c")
    s = lax.axis_index("s")
    wid = s * 2 + c
    cam = wid // 4
    lb = (wid % 4) * 256
    for ch in range(2):
        l0 = lb + ch * 128
        k0 = cam * NIDP + l0
        pltpu.sync_copy(win_hbm.at[pl.ds(k0, 128)], win_v)
        for v in range(8):
            w = win_v[pl.ds(v * 16, 16)]
            kv = k0 + v * 16 + lax.iota(jnp.int32, 16)
            has = w >= 0.0
            wi = w.astype(jnp.int32)
            idx1_v[pl.ds(v * 16, 16)] = jnp.where(has, 2 * kv + 1, 2 * kv)
            jz = FBLK + T2_BLK * (kv & 7) + ((kv >> 3) & 63)
            idx2_v[pl.ds(v * 16, 16)] = jnp.where(has, wi, jz)
        pltpu.async_copy(t1_hbm.at[idx1_v], buf, sem).wait()
        pltpu.async_copy(t2_hbm.at[idx2_v], buf, sem, add=True).wait()
        r0 = cam * NID + l0
        last = jnp.logical_and(wid % 4 == 3, ch == 1)

        @pl.when(jnp.logical_not(last))
        def _():
            pltpu.sync_copy(buf, out_hbm.at[pl.ds(r0, 128)])

        @pl.when(last)
        def _():
            pltpu.sync_copy(buf.at[pl.ds(0, NID - 896)],
                            out_hbm.at[pl.ds(r0, NID - 896)])


def _build_tables(intra_anchors, features):
    return pl.pallas_call(
        _norm_tables_kernel,
        grid=(NCAM,),
        in_specs=[
            pl.BlockSpec((1, NID, D), lambda i: (i, 0, 0)),
            pl.BlockSpec((FBLK, D), lambda i: (i, 0)),
        ],
        out_specs=[
            pl.BlockSpec((1, NIDP, 2 * D), lambda i: (i, 0, 0)),
            pl.BlockSpec((1, 1, NIDP), lambda i: (i, 0, 0)),
            pl.BlockSpec((FBLK, D), lambda i: (i, 0)),
            pl.BlockSpec((T2_BLK, D), lambda i: (i, 0)),
        ],
        out_shape=[
            jax.ShapeDtypeStruct((NCAM, NIDP, 2 * D), jnp.float32),
            jax.ShapeDtypeStruct((NCAM, 1, NIDP), jnp.float32),
            jax.ShapeDtypeStruct((N, D), jnp.float32),
            jax.ShapeDtypeStruct((T2_ROWS, D), jnp.float32),
        ],
    )(intra_anchors, features)


def _winner(keys_f):
    return pl.pallas_call(
        _winner_kernel,
        grid=(NCAM * NIDP // 128,),
        in_specs=[pl.BlockSpec((N, 1), lambda b: (0, 0))],
        out_specs=pl.BlockSpec((1, 1, 128), lambda b: (b, 0, 0)),
        out_shape=jax.ShapeDtypeStruct((NCAM * NIDP // 128, 1, 128),
                                       jnp.float32),
    )(keys_f)


def _loss_sum(t1, yy, fn, labf, camf):
    return pl.pallas_call(
        _loss_kernel,
        grid=(NCAM, N // SBLK),
        in_specs=[
            pl.BlockSpec((1, NIDP, D), lambda i, j: (i, 0, 0)),
            pl.BlockSpec((1, 1, NIDP), lambda i, j: (i, 0, 0)),
            pl.BlockSpec((SBLK, D), lambda i, j: (j, 0)),
            pl.BlockSpec((SBLK, 1), lambda i, j: (j, 0)),
            pl.BlockSpec((SBLK, 1), lambda i, j: (j, 0)),
        ],
        out_specs=pl.BlockSpec((1, 1), lambda i, j: (0, 0),
                               memory_space=pltpu.SMEM),
        out_shape=jax.ShapeDtypeStruct((1, 1), jnp.float32),
    )(t1, yy, fn, labf, camf)


def _update_anchors(winner_flat, t1_flat, t2):
    mesh = plsc.VectorSubcoreMesh(core_axis_name="c", subcore_axis_name="s")
    run = pl.kernel(
        _update_kernel,
        out_type=jax.ShapeDtypeStruct((NCAM * NID, D), jnp.float32),
        mesh=mesh,
        scratch_types=[
            pltpu.VMEM((128,), jnp.float32),
            pltpu.VMEM((128,), jnp.int32),
            pltpu.VMEM((128,), jnp.int32),
            pltpu.VMEM((128, D), jnp.float32),
            pltpu.SemaphoreType.DMA,
        ],
    )
    return run(winner_flat, t1_flat, t2)


def kernel(features, labels, cams, intra_anchors, cross_anchors, epoch):
    labels0 = (labels - 1).astype(jnp.int32)
    cams0 = (cams - 1).astype(jnp.int32)
    keys_f = (cams0 * NIDP + labels0).astype(jnp.float32).reshape(N, 1)
    labf = labels0.astype(jnp.float32).reshape(N, 1)
    camf = cams0.astype(jnp.float32).reshape(N, 1)

    t1, yy, fn, t2 = _build_tables(intra_anchors, features)

    def warm_fn(_):
        winner = _winner(keys_f).reshape(NCAM * NIDP)
        loss_sum = _loss_sum(t1, yy, fn, labf, camf)
        out = _update_anchors(winner, t1.reshape(2 * NCAM * NIDP, D), t2)
        loss = loss_sum[0, 0] * jnp.float32(1.0 / N)
        return loss, out.reshape(NCAM, NID, D)

    def cold_fn(_):
        return jnp.float32(0.0), t1[:, :NID, :D]

    warm = epoch <= WARM_UP_EPOCHS
    loss, new_anchors = lax.cond(warm, warm_fn, cold_fn, 0)
    return (loss, new_anchors, cross_anchors)


# speedup vs baseline: 14.7640x; 14.7640x over previous
import jax
import jax.numpy as jnp
from jax import lax
from jax.experimental import pallas as pl
from jax.experimental.pallas import tpu as pltpu
from jax.experimental.pallas import tpu_sc as plsc

NCAM = 8
NID = 1000
NIDP = 1024
D = 512
N = 4096
MARGIN = 0.3
WARM_UP_EPOCHS = 5

FBLK = 512
SBLK = 128
T2_ROWS = N + 512
T2_BLK = T2_ROWS // NCAM


def _norm_tables_kernel(a_ref, f_ref, an_ref, yy_ref, fn_ref, t2_ref):
    a = a_ref[0]
    ss = jnp.sum(a * a, axis=1, keepdims=True)
    an = a / (jnp.sqrt(ss) + 1e-12)
    an_ref[0, :NID, :] = an
    an_ref[0, NID:, :] = jnp.zeros((NIDP - NID, D), jnp.float32)
    anp = an_ref[0]
    ones = jnp.ones((1, D), jnp.float32)
    yy = lax.dot_general(ones, anp * anp, (((1,), (1,)), ((), ())),
                         preferred_element_type=jnp.float32)
    colid = lax.broadcasted_iota(jnp.int32, (1, NIDP), 1)
    yy_ref[0] = jnp.where(colid >= NID, jnp.float32(1e9), yy)
    f = f_ref[...]
    fss = jnp.sum(f * f, axis=1, keepdims=True)
    fn = f / (jnp.sqrt(fss) + 1e-12)
    fn_ref[...] = fn
    t2_ref[:FBLK, :] = 0.5 * fn
    t2_ref[FBLK:, :] = jnp.zeros((T2_BLK - FBLK, D), jnp.float32)


def _winner_kernel(k_ref, w_ref):
    b = pl.program_id(0)
    kv = (b * 128 + lax.broadcasted_iota(jnp.int32, (1, 128), 1)
          ).astype(jnp.float32)
    keys = k_ref[...]
    eq = keys == kv
    nidx = lax.broadcasted_iota(jnp.int32, (N, 128), 0).astype(jnp.float32)
    cand = jnp.where(eq, nidx, jnp.float32(-1.0))
    w_ref[0] = jnp.max(cand, axis=0, keepdims=True)


def _loss_kernel(an_ref, yy_ref, f_ref, lab_ref, cam_ref, loss_ref):
    i = pl.program_id(0)
    j = pl.program_id(1)

    @pl.when(jnp.logical_and(i == 0, j == 0))
    def _():
        loss_ref[0, 0] = jnp.float32(0.0)

    f = f_ref[...]
    a = an_ref[0]
    s = lax.dot_general(f, a, (((1,), (1,)), ((), ())),
                        preferred_element_type=jnp.float32)
    xx = jnp.sum(f * f, axis=1, keepdims=True)
    d = xx + yy_ref[0] - 2.0 * s
    d = jnp.maximum(d, jnp.float32(1e-12))
    lab = lab_ref[...]
    col = lax.broadcasted_iota(jnp.int32, (SBLK, NIDP), 1).astype(jnp.float32)
    oneh = col == lab
    dn = jnp.min(jnp.where(oneh, jnp.float32(1e30), d), axis=1,
                 keepdims=True)
    dp = jnp.sum(jnp.where(oneh, d, jnp.float32(0.0)), axis=1,
                 keepdims=True)
    cam = cam_ref[...]
    hinge = jnp.maximum(dp - dn + jnp.float32(MARGIN), jnp.float32(0.0))
    contrib = jnp.where(cam == i.astype(jnp.float32), hinge,
                        jnp.float32(0.0))
    loss_ref[0, 0] += jnp.sum(contrib)


def _update_kernel(win_hbm, t2_hbm, outb_hbm, win_v, idx_v, buf, sem):
    c = lax.axis_index("c")
    s = lax.axis_index("s")
    wid = s * 2 + c
    cam = wid // 4
    lb = (wid % 4) * 256
    for ch in range(2):
        l0 = lb + ch * 128
        k0 = cam * NIDP + l0
        pltpu.sync_copy(win_hbm.at[pl.ds(k0, 128)], win_v)
        for v in range(8):
            w = win_v[pl.ds(v * 16, 16)]
            kv = k0 + v * 16 + lax.iota(jnp.int32, 16)
            has = w >= 0.0
            wi = w.astype(jnp.int32)
            wi = wi + (wi >> 9) * (T2_BLK - FBLK)
            jz = FBLK + T2_BLK * (kv & 7) + ((kv >> 3) & 63)
            idx_v[pl.ds(v * 16, 16)] = jnp.where(has, wi, jz)
        pltpu.async_copy(t2_hbm.at[idx_v], buf, sem).wait()
        r0 = cam * NID + l0
        last = jnp.logical_and(wid % 4 == 3, ch == 1)

        @pl.when(jnp.logical_not(last))
        def _():
            pltpu.sync_copy(buf, outb_hbm.at[pl.ds(r0, 128)])

        @pl.when(last)
        def _():
            pltpu.sync_copy(buf.at[pl.ds(0, NID - 896)],
                            outb_hbm.at[pl.ds(r0, NID - 896)])


def _combine_kernel(an_ref, win_ref, b_ref, out_ref):
    a = an_ref[0]
    w = win_ref[0]
    scale = jnp.where(w >= 0.0, jnp.float32(0.5), jnp.float32(1.0))
    rowa = a * scale
    out_ref[0] = rowa[:NID, :] + b_ref[...]


def _build_tables(intra_anchors, features):
    return pl.pallas_call(
        _norm_tables_kernel,
        grid=(NCAM,),
        in_specs=[
            pl.BlockSpec((1, NID, D), lambda i: (i, 0, 0)),
            pl.BlockSpec((FBLK, D), lambda i: (i, 0)),
        ],
        out_specs=[
            pl.BlockSpec((1, NIDP, D), lambda i: (i, 0, 0)),
            pl.BlockSpec((1, 1, NIDP), lambda i: (i, 0, 0)),
            pl.BlockSpec((FBLK, D), lambda i: (i, 0)),
            pl.BlockSpec((T2_BLK, D), lambda i: (i, 0)),
        ],
        out_shape=[
            jax.ShapeDtypeStruct((NCAM, NIDP, D), jnp.float32),
            jax.ShapeDtypeStruct((NCAM, 1, NIDP), jnp.float32),
            jax.ShapeDtypeStruct((N, D), jnp.float32),
            jax.ShapeDtypeStruct((T2_ROWS, D), jnp.float32),
        ],
    )(intra_anchors, features)


def _winner(keys_f):
    return pl.pallas_call(
        _winner_kernel,
        grid=(NCAM * NIDP // 128,),
        in_specs=[pl.BlockSpec((N, 1), lambda b: (0, 0))],
        out_specs=pl.BlockSpec((1, 1, 128), lambda b: (b, 0, 0)),
        out_shape=jax.ShapeDtypeStruct((NCAM * NIDP // 128, 1, 128),
                                       jnp.float32),
    )(keys_f)


def _loss_sum(ann, yy, fn, labf, camf):
    return pl.pallas_call(
        _loss_kernel,
        grid=(NCAM, N // SBLK),
        in_specs=[
            pl.BlockSpec((1, NIDP, D), lambda i, j: (i, 0, 0)),
            pl.BlockSpec((1, 1, NIDP), lambda i, j: (i, 0, 0)),
            pl.BlockSpec((SBLK, D), lambda i, j: (j, 0)),
            pl.BlockSpec((SBLK, 1), lambda i, j: (j, 0)),
            pl.BlockSpec((SBLK, 1), lambda i, j: (j, 0)),
        ],
        out_specs=pl.BlockSpec((1, 1), lambda i, j: (0, 0),
                               memory_space=pltpu.SMEM),
        out_shape=jax.ShapeDtypeStruct((1, 1), jnp.float32),
    )(ann, yy, fn, labf, camf)


def _update_rows(winner_flat, t2):
    mesh = plsc.VectorSubcoreMesh(core_axis_name="c", subcore_axis_name="s")
    run = pl.kernel(
        _update_kernel,
        out_type=jax.ShapeDtypeStruct((NCAM * NID, D), jnp.float32),
        mesh=mesh,
        scratch_types=[
            pltpu.VMEM((128,), jnp.float32),
            pltpu.VMEM((128,), jnp.int32),
            pltpu.VMEM((128, D), jnp.float32),
            pltpu.SemaphoreType.DMA,
        ],
    )
    return run(winner_flat, t2)


def _combine(ann, winner_col, outb):
    return pl.pallas_call(
        _combine_kernel,
        grid=(NCAM,),
        in_specs=[
            pl.BlockSpec((1, NIDP, D), lambda i: (i, 0, 0)),
            pl.BlockSpec((1, NIDP, 1), lambda i: (i, 0, 0)),
            pl.BlockSpec((NID, D), lambda i: (i, 0)),
        ],
        out_specs=pl.BlockSpec((1, NID, D), lambda i: (i, 0, 0)),
        out_shape=jax.ShapeDtypeStruct((NCAM, NID, D), jnp.float32),
    )(ann, winner_col, outb)


def kernel(features, labels, cams, intra_anchors, cross_anchors, epoch):
    labels0 = (labels - 1).astype(jnp.int32)
    cams0 = (cams - 1).astype(jnp.int32)
    keys_f = (cams0 * NIDP + labels0).astype(jnp.float32).reshape(N, 1)
    labf = labels0.astype(jnp.float32).reshape(N, 1)
    camf = cams0.astype(jnp.float32).reshape(N, 1)

    ann, yy, fn, t2 = _build_tables(intra_anchors, features)

    def warm_fn(_):
        winner = _winner(keys_f)
        loss_sum = _loss_sum(ann, yy, fn, labf, camf)
        outb = _update_rows(winner.reshape(NCAM * NIDP), t2)
        new_anchors = _combine(ann, winner.reshape(NCAM, NIDP, 1), outb)
        loss = loss_sum[0, 0] * jnp.float32(1.0 / N)
        return loss, new_anchors

    def cold_fn(_):
        return jnp.float32(0.0), ann[:, :NID, :]

    warm = epoch <= WARM_UP_EPOCHS
    loss, new_anchors = lax.cond(warm, warm_fn, cold_fn, 0)
    return (loss, new_anchors, cross_anchors)


# speedup vs baseline: 15.5200x; 1.0512x over previous
import jax
import jax.numpy as jnp
from jax import lax
from jax.experimental import pallas as pl
from jax.experimental.pallas import tpu as pltpu
from jax.experimental.pallas import tpu_sc as plsc

NCAM = 8
NID = 1000
NIDP = 1024
D = 512
N = 4096
MARGIN = 0.3
WARM_UP_EPOCHS = 5

FBLK = 512
SBLK = 128
T2_ROWS = N + 512
T2_BLK = T2_ROWS // NCAM


def _norm_tables_kernel(a_ref, f_ref, an_ref, yy_ref, fn_ref, t2_ref,
                        anb_ref, fnb_ref, xx_ref):
    a = a_ref[0]
    ss = jnp.sum(a * a, axis=1, keepdims=True)
    an = a / (jnp.sqrt(ss) + 1e-12)
    an_ref[0, :NID, :] = an
    an_ref[0, NID:, :] = jnp.zeros((NIDP - NID, D), jnp.float32)
    anp = an_ref[0]
    ones = jnp.ones((1, D), jnp.float32)
    yy = lax.dot_general(ones, anp * anp, (((1,), (1,)), ((), ())),
                         preferred_element_type=jnp.float32)
    colid = lax.broadcasted_iota(jnp.int32, (1, NIDP), 1)
    yy_ref[0] = jnp.where(colid >= NID, jnp.float32(1e9), yy)
    anb_ref[0] = anp.astype(jnp.bfloat16)
    f = f_ref[...]
    fss = jnp.sum(f * f, axis=1, keepdims=True)
    fn = f / (jnp.sqrt(fss) + 1e-12)
    fn_ref[...] = fn
    fnb_ref[...] = fn.astype(jnp.bfloat16)
    xx_ref[...] = jnp.sum(fn * fn, axis=1, keepdims=True)
    t2_ref[:FBLK, :] = 0.5 * fn
    t2_ref[FBLK:, :] = jnp.zeros((T2_BLK - FBLK, D), jnp.float32)


def _winner_kernel(k_ref, w_ref):
    b = pl.program_id(0)
    kv = (b * 128 + lax.broadcasted_iota(jnp.int32, (1, 128), 1)
          ).astype(jnp.float32)
    keys = k_ref[...]
    eq = keys == kv
    nidx = lax.broadcasted_iota(jnp.int32, (N, 128), 0).astype(jnp.float32)
    cand = jnp.where(eq, nidx, jnp.float32(-1.0))
    w_ref[0] = jnp.max(cand, axis=0, keepdims=True)


def _loss_kernel(an_ref, yy_ref, f_ref, xx_ref, lab_ref, cam_ref, loss_ref):
    i = pl.program_id(0)
    j = pl.program_id(1)

    @pl.when(jnp.logical_and(i == 0, j == 0))
    def _():
        loss_ref[0, 0] = jnp.float32(0.0)

    f = f_ref[...]
    a = an_ref[0]
    s = lax.dot_general(f, a, (((1,), (1,)), ((), ())),
                        preferred_element_type=jnp.float32)
    xx = xx_ref[...]
    d = xx + yy_ref[0] - 2.0 * s
    d = jnp.maximum(d, jnp.float32(1e-12))
    lab = lab_ref[...]
    col = lax.broadcasted_iota(jnp.int32, (SBLK, NIDP), 1).astype(jnp.float32)
    oneh = col == lab
    dn = jnp.min(jnp.where(oneh, jnp.float32(1e30), d), axis=1,
                 keepdims=True)
    dp = jnp.sum(jnp.where(oneh, d, jnp.float32(0.0)), axis=1,
                 keepdims=True)
    cam = cam_ref[...]
    hinge = jnp.maximum(dp - dn + jnp.float32(MARGIN), jnp.float32(0.0))
    contrib = jnp.where(cam == i.astype(jnp.float32), hinge,
                        jnp.float32(0.0))
    loss_ref[0, 0] += jnp.sum(contrib)


def _update_kernel(win_hbm, t2_hbm, outb_hbm, win_v, idx_v, buf, sem):
    c = lax.axis_index("c")
    s = lax.axis_index("s")
    wid = s * 2 + c
    cam = wid // 4
    lb = (wid % 4) * 256
    for ch in range(2):
        l0 = lb + ch * 128
        k0 = cam * NIDP + l0
        pltpu.sync_copy(win_hbm.at[pl.ds(k0, 128)], win_v)
        for v in range(8):
            w = win_v[pl.ds(v * 16, 16)]
            kv = k0 + v * 16 + lax.iota(jnp.int32, 16)
            has = w >= 0.0
            wi = w.astype(jnp.int32)
            wi = wi + (wi >> 9) * (T2_BLK - FBLK)
            jz = FBLK + T2_BLK * (kv & 7) + ((kv >> 3) & 63)
            idx_v[pl.ds(v * 16, 16)] = jnp.where(has, wi, jz)
        pltpu.async_copy(t2_hbm.at[idx_v], buf, sem).wait()
        r0 = cam * NID + l0
        last = jnp.logical_and(wid % 4 == 3, ch == 1)

        @pl.when(jnp.logical_not(last))
        def _():
            pltpu.sync_copy(buf, outb_hbm.at[pl.ds(r0, 128)])

        @pl.when(last)
        def _():
            pltpu.sync_copy(buf.at[pl.ds(0, NID - 896)],
                            outb_hbm.at[pl.ds(r0, NID - 896)])


def _combine_kernel(an_ref, win_ref, b_ref, out_ref):
    a = an_ref[0]
    w = win_ref[0]
    scale = jnp.where(w >= 0.0, jnp.float32(0.5), jnp.float32(1.0))
    rowa = a * scale
    out_ref[0] = rowa[:NID, :] + b_ref[...]


def _build_tables(intra_anchors, features):
    return pl.pallas_call(
        _norm_tables_kernel,
        grid=(NCAM,),
        in_specs=[
            pl.BlockSpec((1, NID, D), lambda i: (i, 0, 0)),
            pl.BlockSpec((FBLK, D), lambda i: (i, 0)),
        ],
        out_specs=[
            pl.BlockSpec((1, NIDP, D), lambda i: (i, 0, 0)),
            pl.BlockSpec((1, 1, NIDP), lambda i: (i, 0, 0)),
            pl.BlockSpec((FBLK, D), lambda i: (i, 0)),
            pl.BlockSpec((T2_BLK, D), lambda i: (i, 0)),
            pl.BlockSpec((1, NIDP, D), lambda i: (i, 0, 0)),
            pl.BlockSpec((FBLK, D), lambda i: (i, 0)),
            pl.BlockSpec((FBLK, 1), lambda i: (i, 0)),
        ],
        out_shape=[
            jax.ShapeDtypeStruct((NCAM, NIDP, D), jnp.float32),
            jax.ShapeDtypeStruct((NCAM, 1, NIDP), jnp.float32),
            jax.ShapeDtypeStruct((N, D), jnp.float32),
            jax.ShapeDtypeStruct((T2_ROWS, D), jnp.float32),
            jax.ShapeDtypeStruct((NCAM, NIDP, D), jnp.bfloat16),
            jax.ShapeDtypeStruct((N, D), jnp.bfloat16),
            jax.ShapeDtypeStruct((N, 1), jnp.float32),
        ],
    )(intra_anchors, features)


def _winner(keys_f):
    return pl.pallas_call(
        _winner_kernel,
        grid=(NCAM * NIDP // 128,),
        in_specs=[pl.BlockSpec((N, 1), lambda b: (0, 0))],
        out_specs=pl.BlockSpec((1, 1, 128), lambda b: (b, 0, 0)),
        out_shape=jax.ShapeDtypeStruct((NCAM * NIDP // 128, 1, 128),
                                       jnp.float32),
    )(keys_f)


def _loss_sum(anb, yy, fnb, xx, labf, camf):
    return pl.pallas_call(
        _loss_kernel,
        grid=(NCAM, N // SBLK),
        in_specs=[
            pl.BlockSpec((1, NIDP, D), lambda i, j: (i, 0, 0)),
            pl.BlockSpec((1, 1, NIDP), lambda i, j: (i, 0, 0)),
            pl.BlockSpec((SBLK, D), lambda i, j: (j, 0)),
            pl.BlockSpec((SBLK, 1), lambda i, j: (j, 0)),
            pl.BlockSpec((SBLK, 1), lambda i, j: (j, 0)),
            pl.BlockSpec((SBLK, 1), lambda i, j: (j, 0)),
        ],
        out_specs=pl.BlockSpec((1, 1), lambda i, j: (0, 0),
                               memory_space=pltpu.SMEM),
        out_shape=jax.ShapeDtypeStruct((1, 1), jnp.float32),
    )(anb, yy, fnb, xx, labf, camf)


def _update_rows(winner_flat, t2):
    mesh = plsc.VectorSubcoreMesh(core_axis_name="c", subcore_axis_name="s")
    run = pl.kernel(
        _update_kernel,
        out_type=jax.ShapeDtypeStruct((NCAM * NID, D), jnp.float32),
        mesh=mesh,
        scratch_types=[
            pltpu.VMEM((128,), jnp.float32),
            pltpu.VMEM((128,), jnp.int32),
            pltpu.VMEM((128, D), jnp.float32),
            pltpu.SemaphoreType.DMA,
        ],
    )
    return run(winner_flat, t2)


def _combine(ann, winner_col, outb):
    return pl.pallas_call(
        _combine_kernel,
        grid=(NCAM,),
        in_specs=[
            pl.BlockSpec((1, NIDP, D), lambda i: (i, 0, 0)),
            pl.BlockSpec((1, NIDP, 1), lambda i: (i, 0, 0)),
            pl.BlockSpec((NID, D), lambda i: (i, 0)),
        ],
        out_specs=pl.BlockSpec((1, NID, D), lambda i: (i, 0, 0)),
        out_shape=jax.ShapeDtypeStruct((NCAM, NID, D), jnp.float32),
    )(ann, winner_col, outb)


def kernel(features, labels, cams, intra_anchors, cross_anchors, epoch):
    labels0 = (labels - 1).astype(jnp.int32)
    cams0 = (cams - 1).astype(jnp.int32)
    keys_f = (cams0 * NIDP + labels0).astype(jnp.float32).reshape(N, 1)
    labf = labels0.astype(jnp.float32).reshape(N, 1)
    camf = cams0.astype(jnp.float32).reshape(N, 1)

    ann, yy, fn, t2, anb, fnb, xx = _build_tables(intra_anchors, features)

    def warm_fn(_):
        winner = _winner(keys_f)
        loss_sum = _loss_sum(anb, yy, fnb, xx, labf, camf)
        outb = _update_rows(winner.reshape(NCAM * NIDP), t2)
        new_anchors = _combine(ann, winner.reshape(NCAM, NIDP, 1), outb)
        loss = loss_sum[0, 0] * jnp.float32(1.0 / N)
        return loss, new_anchors

    def cold_fn(_):
        return jnp.float32(0.0), ann[:, :NID, :]

    warm = epoch <= WARM_UP_EPOCHS
    loss, new_anchors = lax.cond(warm, warm_fn, cold_fn, 0)
    return (loss, new_anchors, cross_anchors)


# speedup vs baseline: 28.0691x; 1.8086x over previous
import jax
import jax.numpy as jnp
from jax import lax
from jax.experimental import pallas as pl
from jax.experimental.pallas import tpu as pltpu
from jax.experimental.pallas import tpu_sc as plsc

NCAM = 8
NID = 1000
NIDP = 1024
D = 512
N = 4096
MARGIN = 0.3
WARM_UP_EPOCHS = 5

FBLK = 512
SBLK = 512
WBLK = 1024
T2_ROWS = N + 512
T2_BLK = T2_ROWS // NCAM


def _norm_tables_kernel(a_ref, f_ref, an_ref, yy_ref, t2_ref,
                        anb_ref, fnb_ref, xx_ref):
    a = a_ref[0]
    ss = jnp.sum(a * a, axis=1, keepdims=True)
    an = a / (jnp.sqrt(ss) + 1e-12)
    an_ref[0, :NID, :] = an
    an_ref[0, NID:, :] = jnp.zeros((NIDP - NID, D), jnp.float32)
    anp = an_ref[0]
    ones = jnp.ones((1, D), jnp.float32)
    yy = lax.dot_general(ones, anp * anp, (((1,), (1,)), ((), ())),
                         preferred_element_type=jnp.float32)
    colid = lax.broadcasted_iota(jnp.int32, (1, NIDP), 1)
    yy_ref[0] = jnp.where(colid >= NID, jnp.float32(1e9), yy)
    anb_ref[0] = anp.astype(jnp.bfloat16)
    f = f_ref[...]
    fss = jnp.sum(f * f, axis=1, keepdims=True)
    fn = f / (jnp.sqrt(fss) + 1e-12)
    fnb_ref[...] = fn.astype(jnp.bfloat16)
    xx_ref[...] = jnp.sum(fn * fn, axis=1, keepdims=True)
    t2_ref[:FBLK, :] = 0.5 * fn
    t2_ref[FBLK:, :] = jnp.zeros((T2_BLK - FBLK, D), jnp.float32)


def _winner_kernel(k_ref, w_ref):
    b = pl.program_id(0)
    kv = (b * WBLK + lax.broadcasted_iota(jnp.int32, (1, WBLK), 1)
          ).astype(jnp.float32)
    keys = k_ref[...]
    eq = keys == kv
    nidx = lax.broadcasted_iota(jnp.int32, (N, WBLK), 0).astype(jnp.float32)
    cand = jnp.where(eq, nidx, jnp.float32(-1.0))
    w_ref[0] = jnp.max(cand, axis=0, keepdims=True)


def _loss_kernel(an_ref, yy_ref, f_ref, xx_ref, lab_ref, cam_ref, loss_ref):
    i = pl.program_id(0)
    j = pl.program_id(1)

    @pl.when(jnp.logical_and(i == 0, j == 0))
    def _():
        loss_ref[0, 0] = jnp.float32(0.0)

    f = f_ref[...]
    a = an_ref[0]
    s = lax.dot_general(f, a, (((1,), (1,)), ((), ())),
                        preferred_element_type=jnp.float32)
    xx = xx_ref[...]
    d = xx + yy_ref[0] - 2.0 * s
    d = jnp.maximum(d, jnp.float32(1e-12))
    lab = lab_ref[...]
    col = lax.broadcasted_iota(jnp.int32, (SBLK, NIDP), 1).astype(jnp.float32)
    oneh = col == lab
    dn = jnp.min(jnp.where(oneh, jnp.float32(1e30), d), axis=1,
                 keepdims=True)
    dp = jnp.sum(jnp.where(oneh, d, jnp.float32(0.0)), axis=1,
                 keepdims=True)
    cam = cam_ref[...]
    hinge = jnp.maximum(dp - dn + jnp.float32(MARGIN), jnp.float32(0.0))
    contrib = jnp.where(cam == i.astype(jnp.float32), hinge,
                        jnp.float32(0.0))
    loss_ref[0, 0] += jnp.sum(contrib)


def _update_kernel(win_hbm, t2_hbm, outb_hbm, win_v, idx_v, buf, sem):
    c = lax.axis_index("c")
    s = lax.axis_index("s")
    wid = s * 2 + c
    cam = wid // 4
    lb = (wid % 4) * 256
    for ch in range(2):
        l0 = lb + ch * 128
        k0 = cam * NIDP + l0
        pltpu.sync_copy(win_hbm.at[pl.ds(k0, 128)], win_v)
        for v in range(8):
            w = win_v[pl.ds(v * 16, 16)]
            kv = k0 + v * 16 + lax.iota(jnp.int32, 16)
            has = w >= 0.0
            wi = w.astype(jnp.int32)
            wi = wi + (wi >> 9) * (T2_BLK - FBLK)
            jz = FBLK + T2_BLK * (kv & 7) + ((kv >> 3) & 63)
            idx_v[pl.ds(v * 16, 16)] = jnp.where(has, wi, jz)
        pltpu.async_copy(t2_hbm.at[idx_v], buf, sem).wait()
        r0 = cam * NID + l0
        last = jnp.logical_and(wid % 4 == 3, ch == 1)

        @pl.when(jnp.logical_not(last))
        def _():
            pltpu.sync_copy(buf, outb_hbm.at[pl.ds(r0, 128)])

        @pl.when(last)
        def _():
            pltpu.sync_copy(buf.at[pl.ds(0, NID - 896)],
                            outb_hbm.at[pl.ds(r0, NID - 896)])


def _combine_kernel(an_ref, win_ref, b_ref, out_ref):
    a = an_ref[0]
    w = win_ref[0]
    scale = jnp.where(w >= 0.0, jnp.float32(0.5), jnp.float32(1.0))
    rowa = a * scale
    out_ref[0] = rowa[:NID, :] + b_ref[...]


def _build_tables(intra_anchors, features):
    return pl.pallas_call(
        _norm_tables_kernel,
        grid=(NCAM,),
        in_specs=[
            pl.BlockSpec((1, NID, D), lambda i: (i, 0, 0)),
            pl.BlockSpec((FBLK, D), lambda i: (i, 0)),
        ],
        out_specs=[
            pl.BlockSpec((1, NIDP, D), lambda i: (i, 0, 0)),
            pl.BlockSpec((1, 1, NIDP), lambda i: (i, 0, 0)),
            pl.BlockSpec((T2_BLK, D), lambda i: (i, 0)),
            pl.BlockSpec((1, NIDP, D), lambda i: (i, 0, 0)),
            pl.BlockSpec((FBLK, D), lambda i: (i, 0)),
            pl.BlockSpec((FBLK, 1), lambda i: (i, 0)),
        ],
        out_shape=[
            jax.ShapeDtypeStruct((NCAM, NIDP, D), jnp.float32),
            jax.ShapeDtypeStruct((NCAM, 1, NIDP), jnp.float32),
            jax.ShapeDtypeStruct((T2_ROWS, D), jnp.float32),
            jax.ShapeDtypeStruct((NCAM, NIDP, D), jnp.bfloat16),
            jax.ShapeDtypeStruct((N, D), jnp.bfloat16),
            jax.ShapeDtypeStruct((N, 1), jnp.float32),
        ],
    )(intra_anchors, features)


def _winner(keys_f):
    return pl.pallas_call(
        _winner_kernel,
        grid=(NCAM * NIDP // WBLK,),
        in_specs=[pl.BlockSpec((N, 1), lambda b: (0, 0))],
        out_specs=pl.BlockSpec((1, 1, WBLK), lambda b: (b, 0, 0)),
        out_shape=jax.ShapeDtypeStruct((NCAM * NIDP // WBLK, 1, WBLK),
                                       jnp.float32),
    )(keys_f)


def _loss_sum(anb, yy, fnb, xx, labf, camf):
    return pl.pallas_call(
        _loss_kernel,
        grid=(NCAM, N // SBLK),
        in_specs=[
            pl.BlockSpec((1, NIDP, D), lambda i, j: (i, 0, 0)),
            pl.BlockSpec((1, 1, NIDP), lambda i, j: (i, 0, 0)),
            pl.BlockSpec((SBLK, D), lambda i, j: (j, 0)),
            pl.BlockSpec((SBLK, 1), lambda i, j: (j, 0)),
            pl.BlockSpec((SBLK, 1), lambda i, j: (j, 0)),
            pl.BlockSpec((SBLK, 1), lambda i, j: (j, 0)),
        ],
        out_specs=pl.BlockSpec((1, 1), lambda i, j: (0, 0),
                               memory_space=pltpu.SMEM),
        out_shape=jax.ShapeDtypeStruct((1, 1), jnp.float32),
    )(anb, yy, fnb, xx, labf, camf)


def _update_rows(winner_flat, t2):
    mesh = plsc.VectorSubcoreMesh(core_axis_name="c", subcore_axis_name="s")
    run = pl.kernel(
        _update_kernel,
        out_type=jax.ShapeDtypeStruct((NCAM * NID, D), jnp.float32),
        mesh=mesh,
        scratch_types=[
            pltpu.VMEM((128,), jnp.float32),
            pltpu.VMEM((128,), jnp.int32),
            pltpu.VMEM((128, D), jnp.float32),
            pltpu.SemaphoreType.DMA,
        ],
    )
    return run(winner_flat, t2)


def _combine(ann, winner_col, outb):
    return pl.pallas_call(
        _combine_kernel,
        grid=(NCAM,),
        in_specs=[
            pl.BlockSpec((1, NIDP, D), lambda i: (i, 0, 0)),
            pl.BlockSpec((1, NIDP, 1), lambda i: (i, 0, 0)),
            pl.BlockSpec((NID, D), lambda i: (i, 0)),
        ],
        out_specs=pl.BlockSpec((1, NID, D), lambda i: (i, 0, 0)),
        out_shape=jax.ShapeDtypeStruct((NCAM, NID, D), jnp.float32),
    )(ann, winner_col, outb)


def kernel(features, labels, cams, intra_anchors, cross_anchors, epoch):
    labels0 = (labels - 1).astype(jnp.int32)
    cams0 = (cams - 1).astype(jnp.int32)
    keys_f = (cams0 * NIDP + labels0).astype(jnp.float32).reshape(N, 1)
    labf = labels0.astype(jnp.float32).reshape(N, 1)
    camf = cams0.astype(jnp.float32).reshape(N, 1)

    ann, yy, t2, anb, fnb, xx = _build_tables(intra_anchors, features)

    def warm_fn(_):
        winner = _winner(keys_f)
        loss_sum = _loss_sum(anb, yy, fnb, xx, labf, camf)
        outb = _update_rows(winner.reshape(NCAM * NIDP), t2)
        new_anchors = _combine(ann, winner.reshape(NCAM, NIDP, 1), outb)
        loss = loss_sum[0, 0] * jnp.float32(1.0 / N)
        return loss, new_anchors

    def cold_fn(_):
        return jnp.float32(0.0), ann[:, :NID, :]

    warm = epoch <= WARM_UP_EPOCHS
    loss, new_anchors = lax.cond(warm, warm_fn, cold_fn, 0)
    return (loss, new_anchors, cross_anchors)
